# baseline (speedup 1.0000x reference)
import functools

import jax
import jax.numpy as jnp
from jax import lax
from jax.experimental import pallas as pl
from jax.experimental.pallas import tpu as pltpu
from jax.experimental.pallas import tpu_sc as plsc

HID = 768
NSL = HID // 16
EPS = 1e-12

NC = 2
NS = 16
NW = NC * NS
TOKENS = 4 * 2048
TPW = TOKENS // NW
CHUNK = 64
NCHUNK = TPW // CHUNK


def _rsqrt(v):
    half = v * 0.5
    y = plsc.bitcast(jnp.int32(0x5F3759DF) - (plsc.bitcast(v, jnp.int32) >> 1),
                     jnp.float32)
    for _ in range(3):
        y = y * (1.5 - half * y * y)
    return y


def _make_kernel():
    mesh = plsc.VectorSubcoreMesh(core_axis_name="c", subcore_axis_name="s")

    @functools.partial(
        pl.kernel,
        mesh=mesh,
        out_type=jax.ShapeDtypeStruct((TOKENS, HID), jnp.float32),
        compiler_params=pltpu.CompilerParams(needs_layout_passes=False),
        scratch_types=[
            pltpu.VMEM((NCHUNK, CHUNK), jnp.int32),
            pltpu.VMEM((TPW,), jnp.int32),
            pltpu.VMEM((2, HID), jnp.float32),
            pltpu.VMEM((HID,), jnp.float32),
            pltpu.VMEM((HID,), jnp.float32),
            pltpu.VMEM((CHUNK, HID), jnp.float32),
            pltpu.VMEM((CHUNK, HID), jnp.float32),
            pltpu.SemaphoreType.DMA,
            pltpu.SemaphoreType.DMA,
        ],
    )
    def emb_ln(word_hbm, ids_hbm, tt_hbm, tte_hbm, scale_hbm, bias_hbm,
               out_hbm, idx_v, tt_v, tte_v, scale_v, bias_v, buf0, buf1,
               sem0, sem1):
        wid = lax.axis_index("s") * NC + lax.axis_index("c")
        base = wid * TPW

        pltpu.sync_copy(ids_hbm.at[wid], idx_v)
        pltpu.sync_copy(tt_hbm.at[wid], tt_v)
        pltpu.sync_copy(tte_hbm, tte_v)
        pltpu.sync_copy(scale_hbm, scale_v)
        pltpu.sync_copy(bias_hbm, bias_v)

        bufs = (buf0, buf1)
        sems = (sem0, sem1)
        lane = lax.iota(jnp.int32, 16)

        def start_chunk(c):
            return pltpu.async_copy(word_hbm.at[idx_v.at[c]], bufs[c % 2],
                                    sems[c % 2])

        def process_chunk(c, pending):
            buf = bufs[c % 2]
            pending.wait()
            nxt = start_chunk(c + 1) if c + 1 < NCHUNK else None

            def row_body(r, carry):
                tok = jnp.full((16,), c * CHUNK + r, jnp.int32)
                ttv = plsc.load_gather(tt_v, [tok])
                acc = jnp.zeros((16,), jnp.float32)
                acc2 = jnp.zeros((16,), jnp.float32)
                ys = []
                for j in range(NSL):
                    sl = pl.ds(j * 16, 16)
                    col = lane + (j * 16)
                    trow = plsc.load_gather(tte_v, [ttv, col])
                    y = buf[r, sl] + trow
                    ys.append(y)
                    acc = acc + y
                    acc2 = acc2 + y * y
                total = jnp.sum(acc)
                total2 = jnp.sum(acc2)
                mean = total * (1.0 / HID)
                var = total2 * (1.0 / HID) - mean * mean
                rv = _rsqrt(jnp.full((16,), var + EPS, jnp.float32))
                mean_v = jnp.full((16,), mean, jnp.float32)
                for j in range(NSL):
                    sl = pl.ds(j * 16, 16)
                    rs = rv * scale_v[sl]
                    buf[r, sl] = (ys[j] - mean_v) * rs + bias_v[sl]
                return carry

            lax.fori_loop(0, CHUNK, row_body, 0)
            pltpu.sync_copy(buf, out_hbm.at[pl.ds(base + c * CHUNK, CHUNK)])
            return nxt

        pending = start_chunk(0)
        for c in range(NCHUNK):
            pending = process_chunk(c, pending)

    return emb_ln


_EMB_LN = _make_kernel()


def kernel(input_ids, token_type_ids, attention_mask, word_embeddings,
           token_type_embeddings, ln_scale, ln_bias):
    del attention_mask
    B, T = input_ids.shape
    ids = input_ids.astype(jnp.int32).reshape(NW, NCHUNK, CHUNK)
    tts = token_type_ids.astype(jnp.int32).reshape(NW, TPW)
    out = _EMB_LN(word_embeddings, ids, tts, token_type_embeddings,
                  ln_scale, ln_bias)
    return out.reshape(B, T, HID)
